# baseline (speedup 1.0000x reference)
import functools

import jax
import jax.numpy as jnp
from jax import lax
from jax.experimental import pallas as pl
from jax.experimental.pallas import tpu as pltpu
from jax.experimental.pallas import tpu_sc as plsc


@functools.lru_cache(maxsize=None)
def _make_gather(V, D, N):
    info = plsc.get_sparse_core_info()
    NC, NS = info.num_cores, info.num_subcores
    NW = NC * NS
    CH = 128
    J = 8
    ROWS = CH * J
    assert N % (NW * ROWS) == 0
    per_w = N // NW
    n_steps = per_w // ROWS

    mesh = plsc.VectorSubcoreMesh(core_axis_name="c", subcore_axis_name="s")

    @functools.partial(
        pl.kernel,
        mesh=mesh,
        out_type=jax.ShapeDtypeStruct((N, D), jnp.float32),
        compiler_params=pltpu.CompilerParams(use_tc_tiling_on_sc=False),
        scratch_types=[
            pltpu.VMEM((J, CH), jnp.int32),
            pltpu.VMEM((ROWS, D), jnp.float32),
            pltpu.SemaphoreType.DMA,
        ],
    )
    def k(table_hbm, idx_hbm, out_hbm, idx_v, rows_v, sem):
        wid = lax.axis_index("s") * NC + lax.axis_index("c")
        row0 = wid * (per_w // CH)

        def body(i, carry):
            c0 = row0 + i * J
            pltpu.sync_copy(idx_hbm.at[pl.ds(c0, J)], idx_v)
            copies = []
            for j in range(J):
                copies.append(
                    pltpu.async_copy(
                        table_hbm.at[idx_v.at[j]],
                        rows_v.at[pl.ds(j * CH, CH)],
                        sem,
                    )
                )
            for c in copies:
                c.wait()
            pltpu.sync_copy(rows_v, out_hbm.at[pl.ds(c0 * CH, ROWS)])
            return carry

        lax.fori_loop(0, n_steps, body, 0)

    return k


def kernel(x, table):
    B, S = x.shape
    V, D = table.shape
    N = B * S
    idx = x.reshape(N // 128, 128).astype(jnp.int32)
    out = _make_gather(V, D, N)(table, idx)
    return out.reshape(B, S, D)


# speedup vs baseline: 1.0179x; 1.0179x over previous
import functools

import jax
import jax.numpy as jnp
from jax import lax
from jax.experimental import pallas as pl
from jax.experimental.pallas import tpu as pltpu
from jax.experimental.pallas import tpu_sc as plsc

CH = 128
J = 5


@functools.lru_cache(maxsize=None)
def _make_gather(V, D, N):
    info = plsc.get_sparse_core_info()
    NC, NS = info.num_cores, info.num_subcores
    NW = NC * NS
    ROWS = CH * J
    assert N % (NW * 2 * ROWS) == 0
    per_w = N // NW
    n_chunks = per_w // CH
    n_steps = per_w // ROWS

    mesh = plsc.VectorSubcoreMesh(core_axis_name="c", subcore_axis_name="s")

    @functools.partial(
        pl.kernel,
        mesh=mesh,
        out_type=jax.ShapeDtypeStruct((N, D), jnp.float32),
        compiler_params=pltpu.CompilerParams(use_tc_tiling_on_sc=False),
        scratch_types=[
            pltpu.VMEM((n_chunks, CH), jnp.int32),
            pltpu.VMEM((2, ROWS, D), jnp.float32),
            pltpu.SemaphoreType.DMA,
            pltpu.SemaphoreType.DMA,
        ],
    )
    def k(table_hbm, idx_hbm, out_hbm, idx_v, rows_v, sem_g, sem_w):
        wid = lax.axis_index("s") * NC + lax.axis_index("c")
        base = wid * per_w

        pltpu.sync_copy(idx_hbm.at[pl.ds(wid * n_chunks, n_chunks)], idx_v)

        def fire_gathers(s, buf):
            for j in range(J):
                pltpu.async_copy(
                    table_hbm.at[idx_v.at[s * J + j]],
                    buf.at[pl.ds(j * CH, CH)],
                    sem_g,
                )

        def drain_gathers(buf):
            pltpu.make_async_copy(out_hbm.at[pl.ds(0, ROWS)], buf, sem_g).wait()

        def fire_writeback(s, buf):
            pltpu.async_copy(buf, out_hbm.at[pl.ds(base + s * ROWS, ROWS)], sem_w)

        def drain_writeback(buf):
            pltpu.make_async_copy(buf, out_hbm.at[pl.ds(0, ROWS)], sem_w).wait()

        buf0 = rows_v.at[0]
        buf1 = rows_v.at[1]

        fire_gathers(0, buf0)

        def body(g, carry):
            s0 = 2 * g

            @pl.when(g > 0)
            def _():
                drain_writeback(buf1)

            fire_gathers(s0 + 1, buf1)
            drain_gathers(buf0)
            fire_writeback(s0, buf0)

            drain_writeback(buf0)

            @pl.when(g < n_steps // 2 - 1)
            def _():
                fire_gathers(s0 + 2, buf0)

            drain_gathers(buf1)
            fire_writeback(s0 + 1, buf1)
            return carry

        lax.fori_loop(0, n_steps // 2, body, 0)
        drain_writeback(buf1)

    return k


def kernel(x, table):
    B, S = x.shape
    V, D = table.shape
    N = B * S
    idx = x.reshape(N // CH, CH).astype(jnp.int32)
    out = _make_gather(V, D, N)(table, idx)
    return out.reshape(B, S, D)
